# baseline (speedup 1.0000x reference)
import jax
import jax.numpy as jnp
from jax.experimental import pallas as pl
from jax.experimental.pallas import tpu as pltpu

_K = 16
_C = 4
_BETA = 500.0
_EPS_SM = 1e-12
_EPS = 1e-07
_HALF_INV_BW_CAM = 100.0
_HALF_INV_BW_LAB = 500.0

_HB = 16


def _nid_from_acc(acc, norm):
    r0 = jax.lax.broadcasted_iota(jnp.int32, acc.shape, 0)
    c0 = jax.lax.broadcasted_iota(jnp.int32, acc.shape, 1)
    diag = (jax.lax.rem(r0, _HB) == jax.lax.rem(c0, _HB))
    md = jnp.where(diag, acc, 0.0)

    pr = jax.lax.broadcasted_iota(jnp.int32, (_K + 1, (_K + 1) * _HB), 1)
    pk = jax.lax.broadcasted_iota(jnp.int32, (_K + 1, (_K + 1) * _HB), 0)
    p_ind = jnp.where(pr // _HB == pk, 1.0, 0.0)

    gr = jax.lax.broadcasted_iota(jnp.int32, ((_C + 1) * _HB, _C + 1), 0)
    gc = jax.lax.broadcasted_iota(jnp.int32, ((_C + 1) * _HB, _C + 1), 1)
    g_ind = jnp.where(gr // _HB == gc, 1.0, 0.0)

    dn = (((1,), (0,)), ((), ()))
    m = jax.lax.dot_general(
        jax.lax.dot_general(p_ind, md, dn, preferred_element_type=jnp.float32,
                            precision=jax.lax.Precision.HIGHEST),
        g_ind, dn, preferred_element_type=jnp.float32,
        precision=jax.lax.Precision.HIGHEST)

    p_cl = m[:_K, :_C] / norm
    p_c = m[:_K, _C:_C + 1] / norm
    p_l = m[_K:_K + 1, :_C] / norm

    p_cl = p_cl / jnp.sum(p_cl)
    p_c = p_c / jnp.sum(p_c)
    p_l = p_l / jnp.sum(p_l)

    outer = p_c * p_l
    log_pcl = jnp.log(p_cl + _EPS)
    mi = jnp.sum(p_cl * (log_pcl - jnp.log(outer + _EPS)))
    h_ent = -jnp.sum(p_cl * log_pcl)
    nid = 1.0 - mi / h_ent
    return (nid - 0.95) * 20.0


def _make_hist_kernel(n_steps, norm):
    def _hist_kernel(cam_ref, lab_ref, out_ref, acc_ref):
        j = pl.program_id(0)

        cam = cam_ref[...]
        lab = lab_ref[...]
        _, _, hb, w = cam.shape

        cam_edges = jax.lax.broadcasted_iota(
            jnp.int32, (_K + 1, 1, 1, 1), 0).astype(jnp.float32) \
            * (_HALF_INV_BW_CAM / _K)
        gscaled = (cam[:, 0] + cam[:, 1] + cam[:, 2]) * (_HALF_INV_BW_CAM / 3.0)
        x_cam = (gscaled[None] - cam_edges).astype(jnp.bfloat16)
        t_cam = jnp.tanh(x_cam)
        t_cam = jnp.sum(t_cam, axis=1)
        a = (t_cam[:_K] - t_cam[1:]) * jnp.bfloat16(0.5)

        m = jnp.max(lab, axis=1, keepdims=True)
        e = jnp.exp((lab - m) * _BETA)
        num = e[:, 1] + 2.0 * e[:, 2] + 3.0 * e[:, 3]
        den = ((e[:, 0] + e[:, 1]) + (e[:, 2] + e[:, 3])) + _EPS_SM
        amax = num / den

        lab_edges = (jax.lax.broadcasted_iota(
            jnp.int32, (_C + 1, 1, 1, 1), 0).astype(jnp.float32) - 0.5) \
            * _HALF_INV_BW_LAB
        t_lab = jnp.tanh(amax[None] * _HALF_INV_BW_LAB - lab_edges)
        t_lab = jnp.sum(t_lab, axis=1)
        l = 0.5 * (t_lab[:_C] - t_lab[1:])

        ones_a = jnp.ones((1, hb, w), dtype=jnp.bfloat16)
        a_aug = jnp.concatenate([a, ones_a], axis=0)
        l_aug = jnp.concatenate(
            [l.astype(jnp.bfloat16), ones_a], axis=0)

        a_rows = a_aug.reshape((_K + 1) * hb, w)
        l_rows = l_aug.reshape((_C + 1) * hb, w)

        m_blk = jax.lax.dot_general(
            a_rows, l_rows, (((1,), (1,)), ((), ())),
            preferred_element_type=jnp.float32,
        )

        @pl.when(j == 0)
        def _():
            acc_ref[...] = jnp.zeros_like(acc_ref)

        acc_ref[...] += m_blk

        @pl.when(j == n_steps - 1)
        def _():
            out_ref[...] = jnp.full(
                (1, 1), _nid_from_acc(acc_ref[...], norm), dtype=jnp.float32)

    return _hist_kernel


@jax.jit
def kernel(camera, label):
    b, _, h, w = camera.shape
    n_strips = h // _HB
    norm = float(b * h * w)

    out = pl.pallas_call(
        _make_hist_kernel(n_strips, norm),
        grid=(n_strips,),
        in_specs=[
            pl.BlockSpec((b, 3, _HB, w), lambda j: (0, 0, j, 0)),
            pl.BlockSpec((b, _C, _HB, w), lambda j: (0, 0, j, 0)),
        ],
        out_specs=pl.BlockSpec((1, 1), lambda j: (0, 0)),
        out_shape=jax.ShapeDtypeStruct((1, 1), jnp.float32),
        scratch_shapes=[
            pltpu.VMEM(((_K + 1) * _HB, (_C + 1) * _HB), jnp.float32),
        ],
        compiler_params=pltpu.CompilerParams(
            dimension_semantics=("arbitrary",),
        ),
    )(camera, label)
    return out[0, 0]


# speedup vs baseline: 1.0803x; 1.0803x over previous
import jax
import jax.numpy as jnp
from jax.experimental import pallas as pl
from jax.experimental.pallas import tpu as pltpu

_K = 16
_C = 4
_BETA = 500.0
_EPS_SM = 1e-12
_EPS = 1e-07
_HALF_INV_BW_CAM = 100.0
_HALF_INV_BW_LAB = 500.0

_HB = 32


def _nid_from_acc(acc, norm):
    r0 = jax.lax.broadcasted_iota(jnp.int32, acc.shape, 0)
    c0 = jax.lax.broadcasted_iota(jnp.int32, acc.shape, 1)
    diag = (jax.lax.rem(r0, _HB) == jax.lax.rem(c0, _HB))
    md = jnp.where(diag, acc, 0.0)

    pr = jax.lax.broadcasted_iota(jnp.int32, (_K + 1, (_K + 1) * _HB), 1)
    pk = jax.lax.broadcasted_iota(jnp.int32, (_K + 1, (_K + 1) * _HB), 0)
    p_ind = jnp.where(pr // _HB == pk, 1.0, 0.0)

    gr = jax.lax.broadcasted_iota(jnp.int32, ((_C + 1) * _HB, _C + 1), 0)
    gc = jax.lax.broadcasted_iota(jnp.int32, ((_C + 1) * _HB, _C + 1), 1)
    g_ind = jnp.where(gr // _HB == gc, 1.0, 0.0)

    dn = (((1,), (0,)), ((), ()))
    m = jax.lax.dot_general(
        jax.lax.dot_general(p_ind, md, dn, preferred_element_type=jnp.float32,
                            precision=jax.lax.Precision.HIGHEST),
        g_ind, dn, preferred_element_type=jnp.float32,
        precision=jax.lax.Precision.HIGHEST)

    p_cl = m[:_K, :_C] / norm
    p_c = m[:_K, _C:_C + 1] / norm
    p_l = m[_K:_K + 1, :_C] / norm

    p_cl = p_cl / jnp.sum(p_cl)
    p_c = p_c / jnp.sum(p_c)
    p_l = p_l / jnp.sum(p_l)

    outer = p_c * p_l
    log_pcl = jnp.log(p_cl + _EPS)
    mi = jnp.sum(p_cl * (log_pcl - jnp.log(outer + _EPS)))
    h_ent = -jnp.sum(p_cl * log_pcl)
    nid = 1.0 - mi / h_ent
    return (nid - 0.95) * 20.0


def _make_hist_kernel(n_steps, norm):
    def _hist_kernel(cam_ref, lab_ref, out_ref, acc_ref):
        j = pl.program_id(0)

        cam = cam_ref[...]
        lab = lab_ref[...]
        _, _, hb, w = cam.shape

        cam_edges = jax.lax.broadcasted_iota(
            jnp.int32, (_K + 1, 1, 1, 1), 0).astype(jnp.float32) \
            * (_HALF_INV_BW_CAM / _K)
        gscaled = (cam[:, 0] + cam[:, 1] + cam[:, 2]) * (_HALF_INV_BW_CAM / 3.0)
        t_cam = jnp.tanh(gscaled[None] - cam_edges)
        t_cam = jnp.sum(t_cam, axis=1)
        a = 0.5 * (t_cam[:_K] - t_cam[1:])

        m = jnp.max(lab, axis=1, keepdims=True)
        e = jnp.exp((lab - m) * _BETA)
        num = e[:, 1] + 2.0 * e[:, 2] + 3.0 * e[:, 3]
        den = ((e[:, 0] + e[:, 1]) + (e[:, 2] + e[:, 3])) + _EPS_SM
        amax = num / den

        lab_edges = (jax.lax.broadcasted_iota(
            jnp.int32, (_C + 1, 1, 1, 1), 0).astype(jnp.float32) - 0.5) \
            * _HALF_INV_BW_LAB
        t_lab = jnp.tanh(amax[None] * _HALF_INV_BW_LAB - lab_edges)
        t_lab = jnp.sum(t_lab, axis=1)
        l = 0.5 * (t_lab[:_C] - t_lab[1:])

        ones_a = jnp.ones((1, hb, w), dtype=jnp.float32)
        a_aug = jnp.concatenate([a, ones_a], axis=0)
        l_aug = jnp.concatenate([l, ones_a], axis=0)

        a_rows = a_aug.reshape((_K + 1) * hb, w).astype(jnp.bfloat16)
        l_rows = l_aug.reshape((_C + 1) * hb, w).astype(jnp.bfloat16)

        m_blk = jax.lax.dot_general(
            a_rows, l_rows, (((1,), (1,)), ((), ())),
            preferred_element_type=jnp.float32,
        )

        @pl.when(j == 0)
        def _():
            acc_ref[...] = jnp.zeros_like(acc_ref)

        acc_ref[...] += m_blk

        @pl.when(j == n_steps - 1)
        def _():
            out_ref[...] = jnp.full(
                (1, 1), _nid_from_acc(acc_ref[...], norm), dtype=jnp.float32)

    return _hist_kernel


@jax.jit
def kernel(camera, label):
    b, _, h, w = camera.shape
    n_strips = h // _HB
    norm = float(b * h * w)

    out = pl.pallas_call(
        _make_hist_kernel(n_strips, norm),
        grid=(n_strips,),
        in_specs=[
            pl.BlockSpec((b, 3, _HB, w), lambda j: (0, 0, j, 0)),
            pl.BlockSpec((b, _C, _HB, w), lambda j: (0, 0, j, 0)),
        ],
        out_specs=pl.BlockSpec((1, 1), lambda j: (0, 0)),
        out_shape=jax.ShapeDtypeStruct((1, 1), jnp.float32),
        scratch_shapes=[
            pltpu.VMEM(((_K + 1) * _HB, (_C + 1) * _HB), jnp.float32),
        ],
        compiler_params=pltpu.CompilerParams(
            dimension_semantics=("arbitrary",),
        ),
    )(camera, label)
    return out[0, 0]


# speedup vs baseline: 1.1625x; 1.0761x over previous
import jax
import jax.numpy as jnp
from jax.experimental import pallas as pl
from jax.experimental.pallas import tpu as pltpu

_K = 16
_C = 4
_BETA = 500.0
_EPS_SM = 1e-12
_EPS = 1e-07
_HALF_INV_BW_CAM = 100.0
_HALF_INV_BW_LAB = 500.0

_HB = 32


def _nid_from_acc(acc, norm):
    r0 = jax.lax.broadcasted_iota(jnp.int32, acc.shape, 0)
    c0 = jax.lax.broadcasted_iota(jnp.int32, acc.shape, 1)
    diag = (jax.lax.rem(r0, _HB) == jax.lax.rem(c0, _HB))
    md = jnp.where(diag, acc, 0.0)

    pr = jax.lax.broadcasted_iota(jnp.int32, (_K + 1, (_K + 1) * _HB), 1)
    pk = jax.lax.broadcasted_iota(jnp.int32, (_K + 1, (_K + 1) * _HB), 0)
    p_ind = jnp.where(pr // _HB == pk, 1.0, 0.0)

    gr = jax.lax.broadcasted_iota(jnp.int32, ((_C + 1) * _HB, _C + 1), 0)
    gc = jax.lax.broadcasted_iota(jnp.int32, ((_C + 1) * _HB, _C + 1), 1)
    g_ind = jnp.where(gr // _HB == gc, 1.0, 0.0)

    dn = (((1,), (0,)), ((), ()))
    m = jax.lax.dot_general(
        jax.lax.dot_general(p_ind, md, dn, preferred_element_type=jnp.float32,
                            precision=jax.lax.Precision.HIGHEST),
        g_ind, dn, preferred_element_type=jnp.float32,
        precision=jax.lax.Precision.HIGHEST)

    p_cl = m[:_K, :_C] / norm
    p_c = m[:_K, _C:_C + 1] / norm
    p_l = m[_K:_K + 1, :_C] / norm

    p_cl = p_cl / jnp.sum(p_cl)
    p_c = p_c / jnp.sum(p_c)
    p_l = p_l / jnp.sum(p_l)

    outer = p_c * p_l
    log_pcl = jnp.log(p_cl + _EPS)
    mi = jnp.sum(p_cl * (log_pcl - jnp.log(outer + _EPS)))
    h_ent = -jnp.sum(p_cl * log_pcl)
    nid = 1.0 - mi / h_ent
    return (nid - 0.95) * 20.0


def _make_hist_kernel(n_steps, norm):
    def _hist_kernel(cam_ref, lab_ref, out_ref, acc_ref):
        j = pl.program_id(0)

        cam = cam_ref[...]
        lab = lab_ref[...]
        _, _, hb, w = cam.shape

        cam_edges = jax.lax.broadcasted_iota(
            jnp.int32, (_K + 1, 1, 1), 0).astype(jnp.float32) \
            * (_HALF_INV_BW_CAM / _K)
        nb = cam.shape[0]
        t_cam = None
        for bi in range(nb):
            gscaled = (cam[bi, 0] + cam[bi, 1] + cam[bi, 2]) \
                * (_HALF_INV_BW_CAM / 3.0)
            t_b = jnp.tanh(gscaled[None] - cam_edges)
            t_cam = t_b if t_cam is None else t_cam + t_b
        a = 0.5 * (t_cam[:_K] - t_cam[1:])

        lab_edges = (jax.lax.broadcasted_iota(
            jnp.int32, (_C + 1, 1, 1), 0).astype(jnp.float32) - 0.5) \
            * _HALF_INV_BW_LAB
        t_lab = None
        for bi in range(nb):
            lb = lab[bi]
            m = jnp.max(lb, axis=0, keepdims=True)
            e = jnp.exp((lb - m) * _BETA)
            num = e[1] + 2.0 * e[2] + 3.0 * e[3]
            den = ((e[0] + e[1]) + (e[2] + e[3])) + _EPS_SM
            amax = num / den
            t_b = jnp.tanh(amax[None] * _HALF_INV_BW_LAB - lab_edges)
            t_lab = t_b if t_lab is None else t_lab + t_b
        l = 0.5 * (t_lab[:_C] - t_lab[1:])

        ones_a = jnp.ones((1, hb, w), dtype=jnp.float32)
        a_aug = jnp.concatenate([a, ones_a], axis=0)
        l_aug = jnp.concatenate([l, ones_a], axis=0)

        a_rows = a_aug.reshape((_K + 1) * hb, w).astype(jnp.bfloat16)
        l_rows = l_aug.reshape((_C + 1) * hb, w).astype(jnp.bfloat16)

        m_blk = jax.lax.dot_general(
            a_rows, l_rows, (((1,), (1,)), ((), ())),
            preferred_element_type=jnp.float32,
        )

        @pl.when(j == 0)
        def _():
            acc_ref[...] = jnp.zeros_like(acc_ref)

        acc_ref[...] += m_blk

        @pl.when(j == n_steps - 1)
        def _():
            out_ref[...] = jnp.full(
                (1, 1), _nid_from_acc(acc_ref[...], norm), dtype=jnp.float32)

    return _hist_kernel


@jax.jit
def kernel(camera, label):
    b, _, h, w = camera.shape
    n_strips = h // _HB
    norm = float(b * h * w)

    out = pl.pallas_call(
        _make_hist_kernel(n_strips, norm),
        grid=(n_strips,),
        in_specs=[
            pl.BlockSpec((b, 3, _HB, w), lambda j: (0, 0, j, 0)),
            pl.BlockSpec((b, _C, _HB, w), lambda j: (0, 0, j, 0)),
        ],
        out_specs=pl.BlockSpec((1, 1), lambda j: (0, 0)),
        out_shape=jax.ShapeDtypeStruct((1, 1), jnp.float32),
        scratch_shapes=[
            pltpu.VMEM(((_K + 1) * _HB, (_C + 1) * _HB), jnp.float32),
        ],
        compiler_params=pltpu.CompilerParams(
            dimension_semantics=("arbitrary",),
        ),
    )(camera, label)
    return out[0, 0]


# speedup vs baseline: 1.1650x; 1.0022x over previous
import jax
import jax.numpy as jnp
from jax.experimental import pallas as pl
from jax.experimental.pallas import tpu as pltpu

_K = 16
_C = 4
_BETA = 500.0
_EPS_SM = 1e-12
_EPS = 1e-07
_HALF_INV_BW_CAM = 100.0
_HALF_INV_BW_LAB = 500.0

_HB = 32


def _nid_from_acc(acc, norm):
    r0 = jax.lax.broadcasted_iota(jnp.int32, acc.shape, 0)
    c0 = jax.lax.broadcasted_iota(jnp.int32, acc.shape, 1)
    diag = (jax.lax.rem(r0, _HB) == jax.lax.rem(c0, _HB))
    md = jnp.where(diag, acc, 0.0)

    pr = jax.lax.broadcasted_iota(jnp.int32, (_K + 1, (_K + 1) * _HB), 1)
    pk = jax.lax.broadcasted_iota(jnp.int32, (_K + 1, (_K + 1) * _HB), 0)
    p_ind = jnp.where(pr // _HB == pk, 1.0, 0.0)

    gr = jax.lax.broadcasted_iota(jnp.int32, ((_C + 1) * _HB, _C + 1), 0)
    gc = jax.lax.broadcasted_iota(jnp.int32, ((_C + 1) * _HB, _C + 1), 1)
    g_ind = jnp.where(gr // _HB == gc, 1.0, 0.0)

    dn = (((1,), (0,)), ((), ()))
    m = jax.lax.dot_general(
        jax.lax.dot_general(p_ind, md, dn, preferred_element_type=jnp.float32,
                            precision=jax.lax.Precision.HIGHEST),
        g_ind, dn, preferred_element_type=jnp.float32,
        precision=jax.lax.Precision.HIGHEST)

    p_cl = m[:_K, :_C] / norm
    p_c = m[:_K, _C:_C + 1] / norm
    p_l = m[_K:_K + 1, :_C] / norm

    p_cl = p_cl / jnp.sum(p_cl)
    p_c = p_c / jnp.sum(p_c)
    p_l = p_l / jnp.sum(p_l)

    outer = p_c * p_l
    log_pcl = jnp.log(p_cl + _EPS)
    mi = jnp.sum(p_cl * (log_pcl - jnp.log(outer + _EPS)))
    h_ent = -jnp.sum(p_cl * log_pcl)
    nid = 1.0 - mi / h_ent
    return (nid - 0.95) * 20.0


def _make_hist_kernel(n_steps, norm):
    def _hist_kernel(cam_ref, lab_ref, out_ref, acc_ref):
        j = pl.program_id(0)

        cam = cam_ref[...]
        lab = lab_ref[...]
        _, _, hb, w = cam.shape

        cam_edges = jax.lax.broadcasted_iota(
            jnp.int32, (_K + 1, 1, 1), 0).astype(jnp.float32) \
            * (_HALF_INV_BW_CAM / _K)
        nb = cam.shape[0]
        t_cam = None
        for bi in range(nb):
            gscaled = (cam[bi, 0] + cam[bi, 1] + cam[bi, 2]) \
                * (_HALF_INV_BW_CAM / 3.0)
            x_b = (gscaled[None] - cam_edges).astype(jnp.bfloat16)
            t_b = jnp.tanh(x_b)
            t_cam = t_b if t_cam is None else t_cam + t_b
        a = (t_cam[:_K] - t_cam[1:]) * jnp.bfloat16(0.5)

        lab_edges = (jax.lax.broadcasted_iota(
            jnp.int32, (_C + 1, 1, 1), 0).astype(jnp.float32) - 0.5) \
            * _HALF_INV_BW_LAB
        t_lab = None
        for bi in range(nb):
            lb = lab[bi]
            m = jnp.max(lb, axis=0, keepdims=True)
            e = jnp.exp((lb - m) * _BETA)
            num = e[1] + 2.0 * e[2] + 3.0 * e[3]
            den = ((e[0] + e[1]) + (e[2] + e[3])) + _EPS_SM
            amax = num / den
            t_b = jnp.tanh(amax[None] * _HALF_INV_BW_LAB - lab_edges)
            t_lab = t_b if t_lab is None else t_lab + t_b
        l = 0.5 * (t_lab[:_C] - t_lab[1:])

        a_aug = jnp.concatenate(
            [a, jnp.ones((1, hb, w), dtype=jnp.bfloat16)], axis=0)
        l_aug = jnp.concatenate(
            [l, jnp.ones((1, hb, w), dtype=jnp.float32)], axis=0)

        a_rows = a_aug.reshape((_K + 1) * hb, w)
        l_rows = l_aug.reshape((_C + 1) * hb, w).astype(jnp.bfloat16)

        m_blk = jax.lax.dot_general(
            a_rows, l_rows, (((1,), (1,)), ((), ())),
            preferred_element_type=jnp.float32,
        )

        @pl.when(j == 0)
        def _():
            acc_ref[...] = jnp.zeros_like(acc_ref)

        acc_ref[...] += m_blk

        @pl.when(j == n_steps - 1)
        def _():
            out_ref[...] = jnp.full(
                (1, 1), _nid_from_acc(acc_ref[...], norm), dtype=jnp.float32)

    return _hist_kernel


@jax.jit
def kernel(camera, label):
    b, _, h, w = camera.shape
    n_strips = h // _HB
    norm = float(b * h * w)

    out = pl.pallas_call(
        _make_hist_kernel(n_strips, norm),
        grid=(n_strips,),
        in_specs=[
            pl.BlockSpec((b, 3, _HB, w), lambda j: (0, 0, j, 0)),
            pl.BlockSpec((b, _C, _HB, w), lambda j: (0, 0, j, 0)),
        ],
        out_specs=pl.BlockSpec((1, 1), lambda j: (0, 0)),
        out_shape=jax.ShapeDtypeStruct((1, 1), jnp.float32),
        scratch_shapes=[
            pltpu.VMEM(((_K + 1) * _HB, (_C + 1) * _HB), jnp.float32),
        ],
        compiler_params=pltpu.CompilerParams(
            dimension_semantics=("arbitrary",),
        ),
    )(camera, label)
    return out[0, 0]
